# Initial kernel scaffold; baseline (speedup 1.0000x reference)
import jax
import jax.numpy as jnp
from jax.experimental import pallas as pl


def kernel(x, edge_index, W1, b1, W2, b2):
    raise NotImplementedError("write your pallas kernel here")


# speedup vs baseline: 18.2849x; 18.2849x over previous
import functools

import jax
import jax.numpy as jnp
from jax import lax
from jax.experimental import pallas as pl
from jax.experimental.pallas import tpu as pltpu
from jax.experimental.pallas import tpu_sc as plsc

N_NODES = 10000
N_PAD = 10240
E_EDGES = 320000
E_PAD = 327680
IN_DIM = 128
HID = 16
OUT_DIM = 3

CHUNK = 128
N_CORES = 2
N_SUB = 16
N_WORKERS = N_CORES * N_SUB
EDGES_PER_TILE = E_PAD // N_WORKERS
CHUNKS_PER_TILE = EDGES_PER_TILE // CHUNK
ROWS_PER_TILE = N_PAD // N_SUB

BLK = 512
GRID = N_PAD // BLK

_MESH = plsc.VectorSubcoreMesh(core_axis_name="c", subcore_axis_name="s")


@functools.partial(
    pl.kernel,
    mesh=_MESH,
    out_type=jax.ShapeDtypeStruct((N_CORES, N_PAD, HID), jnp.float32),
    compiler_params=pltpu.CompilerParams(use_tc_tiling_on_sc=False),
    scratch_types=[
        pltpu.VMEM((CHUNK,), jnp.int32),
        pltpu.VMEM((CHUNK,), jnp.int32),
        pltpu.VMEM((CHUNK, HID), jnp.float32),
        pltpu.VMEM_SHARED((N_PAD, HID), jnp.float32),
        pltpu.SemaphoreType.DMA,
    ],
)
def _msg_pass(table, src, dst, out, src_v, dst_v, rows_v, acc_sh, sem):
    cid = lax.axis_index("c")
    sid = lax.axis_index("s")
    wid = cid * N_SUB + sid

    def _zrow(i, carry):
        rows_v[i, :] = jnp.zeros((HID,), jnp.float32)
        return carry

    lax.fori_loop(0, CHUNK, _zrow, 0)
    for k in range(ROWS_PER_TILE // CHUNK):
        pltpu.sync_copy(
            rows_v, acc_sh.at[pl.ds(sid * ROWS_PER_TILE + k * CHUNK, CHUNK)])
    plsc.subcore_barrier()

    def _body(c, carry):
        base = wid * EDGES_PER_TILE + c * CHUNK
        pltpu.sync_copy(src.at[pl.ds(base, CHUNK)], src_v)
        pltpu.sync_copy(dst.at[pl.ds(base, CHUNK)], dst_v)
        pltpu.async_copy(table.at[src_v], rows_v, sem).wait()
        pltpu.sync_copy(rows_v, acc_sh.at[dst_v], add=True)
        return carry

    lax.fori_loop(0, CHUNKS_PER_TILE, _body, 0)
    plsc.subcore_barrier()

    pltpu.sync_copy(
        acc_sh.at[pl.ds(sid * ROWS_PER_TILE, ROWS_PER_TILE)],
        out.at[cid, pl.ds(sid * ROWS_PER_TILE, ROWS_PER_TILE)])


@functools.partial(
    pl.kernel,
    mesh=_MESH,
    out_type=jax.ShapeDtypeStruct((N_CORES, N_PAD), jnp.float32),
    compiler_params=pltpu.CompilerParams(use_tc_tiling_on_sc=False),
    scratch_types=[
        pltpu.VMEM((CHUNK,), jnp.int32),
        pltpu.VMEM((CHUNK,), jnp.float32),
        pltpu.VMEM_SHARED((N_PAD,), jnp.float32),
        pltpu.SemaphoreType.DMA,
    ],
)
def _degrees(dst, out, dst_v, ones_v, deg_sh, sem):
    cid = lax.axis_index("c")
    sid = lax.axis_index("s")
    wid = cid * N_SUB + sid

    def _fill(val):
        def _f(i, carry):
            ones_v[pl.ds(i * 16, 16)] = jnp.full((16,), val, jnp.float32)
            return carry
        lax.fori_loop(0, CHUNK // 16, _f, 0)

    _fill(0.0)
    for k in range(ROWS_PER_TILE // CHUNK):
        pltpu.sync_copy(
            ones_v, deg_sh.at[pl.ds(sid * ROWS_PER_TILE + k * CHUNK, CHUNK)])
    _fill(1.0)
    plsc.subcore_barrier()

    def _body(c, carry):
        base = wid * EDGES_PER_TILE + c * CHUNK
        pltpu.sync_copy(dst.at[pl.ds(base, CHUNK)], dst_v)
        pltpu.sync_copy(ones_v, deg_sh.at[dst_v], add=True)
        return carry

    lax.fori_loop(0, CHUNKS_PER_TILE, _body, 0)
    plsc.subcore_barrier()

    pltpu.sync_copy(
        deg_sh.at[pl.ds(sid * ROWS_PER_TILE, ROWS_PER_TILE)],
        out.at[cid, pl.ds(sid * ROWS_PER_TILE, ROWS_PER_TILE)])


def _tc_a_body(deg_ref, x_ref, w1_ref, g1_ref, dinv_ref):
    deg = deg_ref[:, 0:1] + deg_ref[:, 1:2] + 1.0
    dinv = lax.rsqrt(deg)
    h = jnp.dot(x_ref[...], w1_ref[...], preferred_element_type=jnp.float32)
    g1_ref[...] = h * dinv
    dinv_ref[...] = dinv


def _tc_b_body(acc_ref, g1_ref, dinv_ref, b1_ref, g2_ref):
    s = acc_ref[0] + acc_ref[1] + g1_ref[...]
    pre = s * dinv_ref[...] + b1_ref[...]
    g2_ref[...] = jnp.maximum(pre, 0.0) * dinv_ref[...]


def _tc_c_body(acc_ref, g2_ref, dinv_ref, w2_ref, b2_ref, out_ref):
    s = (acc_ref[0] + acc_ref[1] + g2_ref[...]) * dinv_ref[...]
    out_ref[...] = (
        jnp.dot(s, w2_ref[...], preferred_element_type=jnp.float32)
        + b2_ref[...])


_tc_a = pl.pallas_call(
    _tc_a_body,
    grid=(GRID,),
    in_specs=[
        pl.BlockSpec((BLK, N_CORES), lambda i: (i, 0)),
        pl.BlockSpec((BLK, IN_DIM), lambda i: (i, 0)),
        pl.BlockSpec((IN_DIM, HID), lambda i: (0, 0)),
    ],
    out_specs=[
        pl.BlockSpec((BLK, HID), lambda i: (i, 0)),
        pl.BlockSpec((BLK, 1), lambda i: (i, 0)),
    ],
    out_shape=[
        jax.ShapeDtypeStruct((N_PAD, HID), jnp.float32),
        jax.ShapeDtypeStruct((N_PAD, 1), jnp.float32),
    ],
)

_tc_b = pl.pallas_call(
    _tc_b_body,
    grid=(GRID,),
    in_specs=[
        pl.BlockSpec((N_CORES, BLK, HID), lambda i: (0, i, 0)),
        pl.BlockSpec((BLK, HID), lambda i: (i, 0)),
        pl.BlockSpec((BLK, 1), lambda i: (i, 0)),
        pl.BlockSpec((1, HID), lambda i: (0, 0)),
    ],
    out_specs=pl.BlockSpec((BLK, HID), lambda i: (i, 0)),
    out_shape=jax.ShapeDtypeStruct((N_PAD, HID), jnp.float32),
)

_tc_c = pl.pallas_call(
    _tc_c_body,
    grid=(GRID,),
    in_specs=[
        pl.BlockSpec((N_CORES, BLK, HID), lambda i: (0, i, 0)),
        pl.BlockSpec((BLK, HID), lambda i: (i, 0)),
        pl.BlockSpec((BLK, 1), lambda i: (i, 0)),
        pl.BlockSpec((HID, OUT_DIM), lambda i: (0, 0)),
        pl.BlockSpec((1, OUT_DIM), lambda i: (0, 0)),
    ],
    out_specs=pl.BlockSpec((BLK, OUT_DIM), lambda i: (i, 0)),
    out_shape=jax.ShapeDtypeStruct((N_PAD, OUT_DIM), jnp.float32),
)


def kernel(x, edge_index, W1, b1, W2, b2):
    src = edge_index[0]
    dst = edge_index[1]
    pad = jnp.full((E_PAD - E_EDGES,), N_NODES, jnp.int32)
    src_p = jnp.concatenate([src, pad])
    dst_p = jnp.concatenate([dst, pad])
    x_p = jnp.pad(x, ((0, N_PAD - N_NODES), (0, 0)))

    deg_parts = _degrees(dst_p)
    g1, dinv = _tc_a(deg_parts.T, x_p, W1)
    acc1 = _msg_pass(g1, src_p, dst_p)
    g2 = _tc_b(acc1, g1, dinv, b1[None, :])
    acc2 = _msg_pass(g2, src_p, dst_p)
    out = _tc_c(acc2, g2, dinv, W2, b2[None, :])
    return out[:N_NODES]


# speedup vs baseline: 35.0814x; 1.9186x over previous
import functools

import jax
import jax.numpy as jnp
from jax import lax
from jax.experimental import pallas as pl
from jax.experimental.pallas import tpu as pltpu
from jax.experimental.pallas import tpu_sc as plsc

N_NODES = 10000
N_PAD = 10240
E_EDGES = 320000
E_PAD = 327680
IN_DIM = 128
HID = 16
OUT_DIM = 3

CHUNK = 128
N_CORES = 2
N_SUB = 16
N_WORKERS = N_CORES * N_SUB
EDGES_PER_TILE = E_PAD // N_WORKERS
CHUNKS_PER_TILE = EDGES_PER_TILE // CHUNK
ROWS_PER_TILE = N_PAD // N_SUB

BLK = 512
GRID = N_PAD // BLK

_MESH = plsc.VectorSubcoreMesh(core_axis_name="c", subcore_axis_name="s")


NBUF = 8
GROUPS = CHUNKS_PER_TILE // NBUF


@functools.partial(
    pl.kernel,
    mesh=_MESH,
    out_type=jax.ShapeDtypeStruct((N_CORES, N_PAD, HID), jnp.float32),
    compiler_params=pltpu.CompilerParams(use_tc_tiling_on_sc=False),
    scratch_types=[
        pltpu.VMEM((CHUNKS_PER_TILE, CHUNK), jnp.int32),
        pltpu.VMEM((CHUNKS_PER_TILE, CHUNK), jnp.int32),
        pltpu.VMEM((NBUF, CHUNK, HID), jnp.float32),
        pltpu.VMEM_SHARED((N_PAD, HID), jnp.float32),
        [pltpu.SemaphoreType.DMA] * NBUF,
    ],
)
def _msg_pass(table, src2d, dst2d, out, src_v, dst_v, rows_v, acc_sh, sems):
    cid = lax.axis_index("c")
    sid = lax.axis_index("s")
    wid = cid * N_SUB + sid

    pltpu.sync_copy(src2d.at[pl.ds(wid * CHUNKS_PER_TILE, CHUNKS_PER_TILE)],
                    src_v)
    pltpu.sync_copy(dst2d.at[pl.ds(wid * CHUNKS_PER_TILE, CHUNKS_PER_TILE)],
                    dst_v)

    def _zrow(i, carry):
        rows_v[0, i, :] = jnp.zeros((HID,), jnp.float32)
        return carry

    lax.fori_loop(0, CHUNK, _zrow, 0)
    for k in range(ROWS_PER_TILE // CHUNK):
        pltpu.sync_copy(
            rows_v.at[0],
            acc_sh.at[pl.ds(sid * ROWS_PER_TILE + k * CHUNK, CHUNK)])
    plsc.subcore_barrier()

    def _gather(c, b):
        pltpu.make_async_copy(
            table.at[src_v.at[c]], rows_v.at[b], sems[b]).start()

    for b in range(NBUF):
        _gather(b, b)

    def _group(g, carry):
        for b in range(NBUF):
            c = g * NBUF + b
            pltpu.make_async_copy(
                table.at[src_v.at[c]], rows_v.at[b], sems[b]).wait()
            pltpu.sync_copy(rows_v.at[b], acc_sh.at[dst_v.at[c]], add=True)

            @pl.when(g < GROUPS - 1)
            def _():
                _gather(c + NBUF, b)
        return carry

    lax.fori_loop(0, GROUPS, _group, 0)
    plsc.subcore_barrier()

    pltpu.sync_copy(
        acc_sh.at[pl.ds(sid * ROWS_PER_TILE, ROWS_PER_TILE)],
        out.at[cid, pl.ds(sid * ROWS_PER_TILE, ROWS_PER_TILE)])


@functools.partial(
    pl.kernel,
    mesh=_MESH,
    out_type=jax.ShapeDtypeStruct((N_CORES, N_PAD), jnp.float32),
    compiler_params=pltpu.CompilerParams(use_tc_tiling_on_sc=False),
    scratch_types=[
        pltpu.VMEM((CHUNKS_PER_TILE, CHUNK), jnp.int32),
        pltpu.VMEM((CHUNK,), jnp.float32),
        pltpu.VMEM_SHARED((N_PAD,), jnp.float32),
        [pltpu.SemaphoreType.DMA] * NBUF,
    ],
)
def _degrees(dst2d, out, dst_v, ones_v, deg_sh, sems):
    cid = lax.axis_index("c")
    sid = lax.axis_index("s")
    wid = cid * N_SUB + sid

    pltpu.sync_copy(dst2d.at[pl.ds(wid * CHUNKS_PER_TILE, CHUNKS_PER_TILE)],
                    dst_v)

    def _fill(val):
        def _f(i, carry):
            ones_v[pl.ds(i * 16, 16)] = jnp.full((16,), val, jnp.float32)
            return carry
        lax.fori_loop(0, CHUNK // 16, _f, 0)

    _fill(0.0)
    for k in range(ROWS_PER_TILE // CHUNK):
        pltpu.sync_copy(
            ones_v, deg_sh.at[pl.ds(sid * ROWS_PER_TILE + k * CHUNK, CHUNK)])
    _fill(1.0)
    plsc.subcore_barrier()

    def _group(g, carry):
        for b in range(NBUF):
            c = g * NBUF + b
            pltpu.async_copy(
                ones_v, deg_sh.at[dst_v.at[c]], sems[b], add=True)
        for b in range(NBUF):
            c = g * NBUF + b
            pltpu.make_async_copy(
                ones_v, deg_sh.at[dst_v.at[c]], sems[b]).wait()
        return carry

    lax.fori_loop(0, GROUPS, _group, 0)
    plsc.subcore_barrier()

    pltpu.sync_copy(
        deg_sh.at[pl.ds(sid * ROWS_PER_TILE, ROWS_PER_TILE)],
        out.at[cid, pl.ds(sid * ROWS_PER_TILE, ROWS_PER_TILE)])


def _tc_a_body(deg_ref, x_ref, w1_ref, g1_ref, dinv_ref):
    deg = deg_ref[:, 0:1] + deg_ref[:, 1:2] + 1.0
    dinv = lax.rsqrt(deg)
    h = jnp.dot(x_ref[...], w1_ref[...], preferred_element_type=jnp.float32)
    g1_ref[...] = h * dinv
    dinv_ref[...] = dinv


def _tc_b_body(acc_ref, g1_ref, dinv_ref, b1_ref, g2_ref):
    s = acc_ref[0] + acc_ref[1] + g1_ref[...]
    pre = s * dinv_ref[...] + b1_ref[...]
    g2_ref[...] = jnp.maximum(pre, 0.0) * dinv_ref[...]


def _tc_c_body(acc_ref, g2_ref, dinv_ref, w2_ref, b2_ref, out_ref):
    s = (acc_ref[0] + acc_ref[1] + g2_ref[...]) * dinv_ref[...]
    out_ref[...] = (
        jnp.dot(s, w2_ref[...], preferred_element_type=jnp.float32)
        + b2_ref[...])


_tc_a = pl.pallas_call(
    _tc_a_body,
    grid=(GRID,),
    in_specs=[
        pl.BlockSpec((BLK, N_CORES), lambda i: (i, 0)),
        pl.BlockSpec((BLK, IN_DIM), lambda i: (i, 0)),
        pl.BlockSpec((IN_DIM, HID), lambda i: (0, 0)),
    ],
    out_specs=[
        pl.BlockSpec((BLK, HID), lambda i: (i, 0)),
        pl.BlockSpec((BLK, 1), lambda i: (i, 0)),
    ],
    out_shape=[
        jax.ShapeDtypeStruct((N_PAD, HID), jnp.float32),
        jax.ShapeDtypeStruct((N_PAD, 1), jnp.float32),
    ],
)

_tc_b = pl.pallas_call(
    _tc_b_body,
    grid=(GRID,),
    in_specs=[
        pl.BlockSpec((N_CORES, BLK, HID), lambda i: (0, i, 0)),
        pl.BlockSpec((BLK, HID), lambda i: (i, 0)),
        pl.BlockSpec((BLK, 1), lambda i: (i, 0)),
        pl.BlockSpec((1, HID), lambda i: (0, 0)),
    ],
    out_specs=pl.BlockSpec((BLK, HID), lambda i: (i, 0)),
    out_shape=jax.ShapeDtypeStruct((N_PAD, HID), jnp.float32),
)

_tc_c = pl.pallas_call(
    _tc_c_body,
    grid=(GRID,),
    in_specs=[
        pl.BlockSpec((N_CORES, BLK, HID), lambda i: (0, i, 0)),
        pl.BlockSpec((BLK, HID), lambda i: (i, 0)),
        pl.BlockSpec((BLK, 1), lambda i: (i, 0)),
        pl.BlockSpec((HID, OUT_DIM), lambda i: (0, 0)),
        pl.BlockSpec((1, OUT_DIM), lambda i: (0, 0)),
    ],
    out_specs=pl.BlockSpec((BLK, OUT_DIM), lambda i: (i, 0)),
    out_shape=jax.ShapeDtypeStruct((N_PAD, OUT_DIM), jnp.float32),
)


def kernel(x, edge_index, W1, b1, W2, b2):
    src = edge_index[0]
    dst = edge_index[1]
    pad = jnp.full((E_PAD - E_EDGES,), N_NODES, jnp.int32)
    src_p = jnp.concatenate([src, pad]).reshape(
        N_WORKERS * CHUNKS_PER_TILE, CHUNK)
    dst_p = jnp.concatenate([dst, pad]).reshape(
        N_WORKERS * CHUNKS_PER_TILE, CHUNK)
    x_p = jnp.pad(x, ((0, N_PAD - N_NODES), (0, 0)))

    deg_parts = _degrees(dst_p)
    g1, dinv = _tc_a(deg_parts.T, x_p, W1)
    acc1 = _msg_pass(g1, src_p, dst_p)
    g2 = _tc_b(acc1, g1, dinv, b1[None, :])
    acc2 = _msg_pass(g2, src_p, dst_p)
    out = _tc_c(acc2, g2, dinv, W2, b2[None, :])
    return out[:N_NODES]
